# baseline (speedup 1.0000x reference)
import jax
import jax.numpy as jnp
from jax.experimental import pallas as pl
from jax.experimental.pallas import tpu as pltpu

V_COARSE = 4096
V_FINE = 16384
COARSE_DIM = 256
FINE_INPUT_DIM = 256
OUTPUT_DIM = 256
SCALAR_PROJ_DIM = 64

ROWS = 2048
KCH = 512
NK = V_COARSE // KCH


def _fused_body(x_coarse_ref, w_sym_ref, b_sym_ref, interp_ref, x_fine_ref,
                w_fuse1_ref, w_fuse2_ref, b_fuse_ref, out_ref,
                xs_ref, acc_ref):
    i = pl.program_id(0)
    k = pl.program_id(1)

    @pl.when(jnp.logical_and(i == 0, k == 0))
    def _():
        xs_ref[...] = (
            jnp.dot(x_coarse_ref[...], w_sym_ref[...],
                    preferred_element_type=jnp.float32)
            + b_sym_ref[...]
        )

    part = jnp.dot(interp_ref[...], xs_ref[pl.ds(k * KCH, KCH), :],
                   preferred_element_type=jnp.float32)

    @pl.when(k == 0)
    def _():
        acc_ref[...] = part

    @pl.when(k > 0)
    def _():
        acc_ref[...] += part

    @pl.when(k == NK - 1)
    def _():
        out_ref[...] = (
            jnp.dot(acc_ref[...], w_fuse1_ref[...],
                    preferred_element_type=jnp.float32)
            + jnp.dot(x_fine_ref[...], w_fuse2_ref[...],
                      preferred_element_type=jnp.float32)
            + b_fuse_ref[...]
        )


def kernel(x_coarse, x_fine_input, interp_matrix, W_sym, b_sym, W_fuse, b_fuse):
    w_fuse1 = W_fuse[:SCALAR_PROJ_DIM, :]
    w_fuse2 = W_fuse[SCALAR_PROJ_DIM:, :]
    b_sym2 = b_sym.reshape(1, SCALAR_PROJ_DIM)
    b_fuse2 = b_fuse.reshape(1, OUTPUT_DIM)

    grid = (V_FINE // ROWS, NK)
    return pl.pallas_call(
        _fused_body,
        grid=grid,
        in_specs=[
            pl.BlockSpec((V_COARSE, COARSE_DIM), lambda i, k: (0, 0)),
            pl.BlockSpec((COARSE_DIM, SCALAR_PROJ_DIM), lambda i, k: (0, 0)),
            pl.BlockSpec((1, SCALAR_PROJ_DIM), lambda i, k: (0, 0)),
            pl.BlockSpec((ROWS, KCH), lambda i, k: (i, k)),
            pl.BlockSpec((ROWS, FINE_INPUT_DIM), lambda i, k: (i, 0)),
            pl.BlockSpec((SCALAR_PROJ_DIM, OUTPUT_DIM), lambda i, k: (0, 0)),
            pl.BlockSpec((FINE_INPUT_DIM, OUTPUT_DIM), lambda i, k: (0, 0)),
            pl.BlockSpec((1, OUTPUT_DIM), lambda i, k: (0, 0)),
        ],
        out_specs=pl.BlockSpec((ROWS, OUTPUT_DIM), lambda i, k: (i, 0)),
        out_shape=jax.ShapeDtypeStruct((V_FINE, OUTPUT_DIM), jnp.float32),
        scratch_shapes=[
            pltpu.VMEM((V_COARSE, SCALAR_PROJ_DIM), jnp.float32),
            pltpu.VMEM((ROWS, SCALAR_PROJ_DIM), jnp.float32),
        ],
        compiler_params=pltpu.CompilerParams(
            dimension_semantics=("arbitrary", "arbitrary")),
    )(x_coarse, W_sym, b_sym2, interp_matrix, x_fine_input,
      w_fuse1, w_fuse2, b_fuse2)


# speedup vs baseline: 1.1818x; 1.1818x over previous
import jax
import jax.numpy as jnp
from jax.experimental import pallas as pl
from jax.experimental.pallas import tpu as pltpu

V_COARSE = 4096
V_FINE = 16384
COARSE_DIM = 256
FINE_INPUT_DIM = 256
OUTPUT_DIM = 256
SCALAR_PROJ_DIM = 64

ROWS = 512
NTILES = V_FINE // ROWS
NBUF = 4
OBUF = 2


def _body(x_coarse_ref, w_sym_ref, b_sym_ref, w_fuse1_ref, w_fuse2_ref,
          b_fuse_ref, interp_hbm, x_fine_hbm, out_hbm,
          xs_ref, ibuf, fbuf, obuf, tmbuf, isem, fsem, osem):
    xs_ref[...] = (
        jnp.dot(x_coarse_ref[...], w_sym_ref[...],
                preferred_element_type=jnp.float32)
        + b_sym_ref[...]
    )

    def icopy(t):
        s = t % NBUF
        return pltpu.make_async_copy(
            interp_hbm.at[pl.ds(t * ROWS, ROWS), :], ibuf.at[s], isem.at[s])

    def fcopy(t):
        s = t % NBUF
        return pltpu.make_async_copy(
            x_fine_hbm.at[pl.ds(t * ROWS, ROWS), :], fbuf.at[s], fsem.at[s])

    def ocopy(t):
        s = t % OBUF
        return pltpu.make_async_copy(
            obuf.at[s], out_hbm.at[pl.ds(t * ROWS, ROWS), :], osem.at[s])

    for t in range(NBUF):
        icopy(t).start()
    for t in range(NBUF - 1):
        fcopy(t).start()

    xs = xs_ref[...]
    wf1 = w_fuse1_ref[...]
    wf2 = w_fuse2_ref[...]
    bf = b_fuse_ref[...]

    def fuse_stage(t):
        if t >= OBUF:
            ocopy(t - OBUF).wait()
        fcopy(t).wait()
        obuf[t % OBUF] = (
            jnp.dot(tmbuf[t % 2], wf1, preferred_element_type=jnp.float32)
            + jnp.dot(fbuf[t % NBUF], wf2, preferred_element_type=jnp.float32)
            + bf
        )
        ocopy(t).start()

    for t in range(NTILES):
        s = t % NBUF
        icopy(t).wait()
        tmbuf[t % 2] = jnp.dot(ibuf[s], xs, preferred_element_type=jnp.float32)
        if t > 0:
            fuse_stage(t - 1)
        if t + NBUF < NTILES:
            icopy(t + NBUF).start()
        if t + NBUF - 1 < NTILES:
            fcopy(t + NBUF - 1).start()
    fuse_stage(NTILES - 1)
    for t in range(NTILES - OBUF, NTILES):
        ocopy(t).wait()


def kernel(x_coarse, x_fine_input, interp_matrix, W_sym, b_sym, W_fuse, b_fuse):
    w_fuse1 = W_fuse[:SCALAR_PROJ_DIM, :]
    w_fuse2 = W_fuse[SCALAR_PROJ_DIM:, :]
    b_sym2 = b_sym.reshape(1, SCALAR_PROJ_DIM)
    b_fuse2 = b_fuse.reshape(1, OUTPUT_DIM)

    vmem = pl.BlockSpec(memory_space=pltpu.MemorySpace.VMEM)
    return pl.pallas_call(
        _body,
        in_specs=[vmem, vmem, vmem, vmem, vmem, vmem,
                  pl.BlockSpec(memory_space=pl.ANY),
                  pl.BlockSpec(memory_space=pl.ANY)],
        out_specs=pl.BlockSpec(memory_space=pl.ANY),
        out_shape=jax.ShapeDtypeStruct((V_FINE, OUTPUT_DIM), jnp.float32),
        scratch_shapes=[
            pltpu.VMEM((V_COARSE, SCALAR_PROJ_DIM), jnp.float32),
            pltpu.VMEM((NBUF, ROWS, V_COARSE), jnp.float32),
            pltpu.VMEM((NBUF, ROWS, FINE_INPUT_DIM), jnp.float32),
            pltpu.VMEM((OBUF, ROWS, OUTPUT_DIM), jnp.float32),
            pltpu.VMEM((2, ROWS, SCALAR_PROJ_DIM), jnp.float32),
            pltpu.SemaphoreType.DMA((NBUF,)),
            pltpu.SemaphoreType.DMA((NBUF,)),
            pltpu.SemaphoreType.DMA((OBUF,)),
        ],
    )(x_coarse, W_sym, b_sym2, w_fuse1, w_fuse2, b_fuse2,
      interp_matrix, x_fine_input)


# speedup vs baseline: 1.2039x; 1.0187x over previous
import jax
import jax.numpy as jnp
from jax.experimental import pallas as pl
from jax.experimental.pallas import tpu as pltpu

V_COARSE = 4096
V_FINE = 16384
COARSE_DIM = 256
FINE_INPUT_DIM = 256
OUTPUT_DIM = 256
SCALAR_PROJ_DIM = 64

ROWS = 512


def _fused_body(x_coarse_ref, w_sym_ref, b_sym_ref, interp_ref, x_fine_ref,
                w_fuse1_ref, w_fuse2_ref, b_fuse_ref, out_ref, x_scalar_ref):
    @pl.when(pl.program_id(0) == 0)
    def _():
        x_scalar_ref[...] = (
            jnp.dot(x_coarse_ref[...], w_sym_ref[...],
                    preferred_element_type=jnp.float32)
            + b_sym_ref[...]
        )

    t = jnp.dot(interp_ref[...], x_scalar_ref[...],
                preferred_element_type=jnp.float32)
    out_ref[...] = (
        jnp.dot(t, w_fuse1_ref[...], preferred_element_type=jnp.float32)
        + jnp.dot(x_fine_ref[...], w_fuse2_ref[...],
                  preferred_element_type=jnp.float32)
        + b_fuse_ref[...]
    )


def kernel(x_coarse, x_fine_input, interp_matrix, W_sym, b_sym, W_fuse, b_fuse):
    w_fuse1 = W_fuse[:SCALAR_PROJ_DIM, :]
    w_fuse2 = W_fuse[SCALAR_PROJ_DIM:, :]
    b_sym2 = b_sym.reshape(1, SCALAR_PROJ_DIM)
    b_fuse2 = b_fuse.reshape(1, OUTPUT_DIM)

    grid = (V_FINE // ROWS,)
    return pl.pallas_call(
        _fused_body,
        grid=grid,
        in_specs=[
            pl.BlockSpec((V_COARSE, COARSE_DIM), lambda i: (0, 0)),
            pl.BlockSpec((COARSE_DIM, SCALAR_PROJ_DIM), lambda i: (0, 0)),
            pl.BlockSpec((1, SCALAR_PROJ_DIM), lambda i: (0, 0)),
            pl.BlockSpec((ROWS, V_COARSE), lambda i: (i, 0)),
            pl.BlockSpec((ROWS, FINE_INPUT_DIM), lambda i: (i, 0)),
            pl.BlockSpec((SCALAR_PROJ_DIM, OUTPUT_DIM), lambda i: (0, 0)),
            pl.BlockSpec((FINE_INPUT_DIM, OUTPUT_DIM), lambda i: (0, 0)),
            pl.BlockSpec((1, OUTPUT_DIM), lambda i: (0, 0)),
        ],
        out_specs=pl.BlockSpec((ROWS, OUTPUT_DIM), lambda i: (i, 0)),
        out_shape=jax.ShapeDtypeStruct((V_FINE, OUTPUT_DIM), jnp.float32),
        scratch_shapes=[pltpu.VMEM((V_COARSE, SCALAR_PROJ_DIM), jnp.float32)],
        compiler_params=pltpu.CompilerParams(
            dimension_semantics=("arbitrary",)),
    )(x_coarse, W_sym, b_sym2, interp_matrix, x_fine_input,
      w_fuse1, w_fuse2, b_fuse2)
